# baseline (speedup 1.0000x reference)
import functools

import jax
import jax.numpy as jnp
from jax import lax
from jax.experimental import pallas as pl
from jax.experimental.pallas import tpu as pltpu

_EPS = 1e-5
_VMEM_LIMIT = 100 * 1024 * 1024


def _fused_block_kernel(x_ref, bw_ref, pk_ref, w3_ref, o_ref, *,
                        half, mid, H, W, bblk):
    L = H * W
    C = x_ref.shape[2]
    f32 = jnp.float32

    dn = (((2,), (0,)), ((), ()))
    y = lax.dot_general(x_ref[...], bw_ref[...], dn,
                        preferred_element_type=f32)
    o_left = y[:, :, :half]
    b1 = pk_ref[0:1, :mid].reshape(1, 1, mid)
    h1 = jnp.maximum(y[:, :, half:] + b1, 0.0)

    zp = jnp.zeros((29, bblk, mid), f32)
    hp = jnp.concatenate([zp, h1, zp], axis=0)
    wco = lax.broadcasted_iota(jnp.int32, (L, 1, 1), 0) % W
    mask_l = (wco != 0).astype(f32)
    mask_r = (wco != W - 1).astype(f32)
    acc = jnp.zeros((L, bblk, mid), f32)
    for dh in (-1, 0, 1):
        for dw in (-1, 0, 1):
            t = 3 * (dh + 1) + (dw + 1)
            tap = hp[29 + dh * W + dw:29 + dh * W + dw + L]
            if dw == -1:
                tap = tap * mask_l
            elif dw == 1:
                tap = tap * mask_r
            acc = acc + tap * pk_ref[4 + t:5 + t, :mid].reshape(1, 1, mid)
    h2 = acc + pk_ref[1:2, :mid].reshape(1, 1, mid)

    out = lax.dot_general(h2, w3_ref[...], dn, preferred_element_type=f32)
    no = C - half
    out = jnp.maximum(out + pk_ref[3:4, :no].reshape(1, 1, no), 0.0)
    o_ref[...] = jnp.concatenate([o_left, out], axis=2)


def _bn_fold(gamma, beta, mean, var):
    s = gamma * lax.rsqrt(var + _EPS)
    return s, beta - mean * s


def kernel(x, channel_choice, bn1_beta, bn1_gamma, bn1_mean, bn1_var,
           bn2_beta, bn2_gamma, bn2_mean, bn2_var,
           bn3_beta, bn3_gamma, bn3_mean, bn3_var,
           w1, w3, wd):
    B, C, H, W = x.shape
    mid = w1.shape[0]
    outputs = w3.shape[0]
    half = C // 2
    L = H * W

    s1, b1 = _bn_fold(bn1_gamma, bn1_beta, bn1_mean, bn1_var)
    s2, b2 = _bn_fold(bn2_gamma, bn2_beta, bn2_mean, bn2_var)
    s3, b3 = _bn_fold(bn3_gamma, bn3_beta, bn3_mean, bn3_var)

    mask = channel_choice[0, :mid]
    w1_eff = w1 * (mask * s1)[:, None]

    left = jnp.stack([jnp.eye(half, dtype=jnp.float32),
                      jnp.zeros((half, half), jnp.float32)],
                     axis=1).reshape(C, half)
    right = jnp.stack([jnp.zeros((half, mid), jnp.float32), w1_eff.T],
                      axis=1).reshape(C, mid)
    big_w = jnp.concatenate([left, right], axis=1)

    pk = jnp.concatenate([
        jnp.stack([b1, b2, s3, b3]),
        wd * s2[None, :],
        jnp.zeros((3, mid), jnp.float32)], axis=0)
    w3_t = (w3 * s3[:, None]).T

    bblk = 8
    xt = x.transpose(2, 3, 0, 1).reshape(L, B, C)
    kern = functools.partial(_fused_block_kernel, half=half, mid=mid, H=H,
                             W=W, bblk=bblk)
    out = pl.pallas_call(
        kern,
        out_shape=jax.ShapeDtypeStruct((L, B, half + outputs), jnp.float32),
        grid_spec=pltpu.PrefetchScalarGridSpec(
            num_scalar_prefetch=0,
            grid=(B // bblk,),
            in_specs=[
                pl.BlockSpec((L, bblk, C), lambda b: (0, b, 0)),
                pl.BlockSpec((C, half + mid), lambda b: (0, 0)),
                pl.BlockSpec((16, mid), lambda b: (0, 0)),
                pl.BlockSpec((mid, outputs), lambda b: (0, 0)),
            ],
            out_specs=pl.BlockSpec((L, bblk, half + outputs),
                                   lambda b: (0, b, 0)),
        ),
        compiler_params=pltpu.CompilerParams(
            dimension_semantics=("parallel",),
            vmem_limit_bytes=_VMEM_LIMIT,
        ),
    )(xt, big_w, pk, w3_t)
    return out.reshape(H, W, B, half + outputs).transpose(2, 3, 0, 1)


# speedup vs baseline: 1.0039x; 1.0039x over previous
import functools

import jax
import jax.numpy as jnp
from jax import lax
from jax.experimental import pallas as pl
from jax.experimental.pallas import tpu as pltpu

_EPS = 1e-5
_VMEM_LIMIT = 100 * 1024 * 1024


def _fused_block_kernel(x_ref, w1_ref, pk_ref, w3_ref, o_ref, *,
                        half, mid, H, W, bblk):
    L = H * W
    C = x_ref.shape[2]
    f32 = jnp.float32

    row = lax.broadcasted_iota(jnp.int32, (C, half), 0)
    col = lax.broadcasted_iota(jnp.int32, (C, half), 1)
    left = (row == 2 * col).astype(f32)
    w1t = w1_ref[...].T
    right = jnp.stack([jnp.zeros((half, mid), f32), w1t],
                      axis=1).reshape(C, mid)
    big_w = jnp.concatenate([left, right], axis=1)

    dn = (((2,), (0,)), ((), ()))
    y = lax.dot_general(x_ref[...], big_w, dn,
                        preferred_element_type=f32)
    o_left = y[:, :, :half]
    b1 = pk_ref[0:1, :mid].reshape(1, 1, mid)
    h1 = jnp.maximum(y[:, :, half:] + b1, 0.0)

    zp = jnp.zeros((29, bblk, mid), f32)
    hp = jnp.concatenate([zp, h1, zp], axis=0)
    wco = lax.broadcasted_iota(jnp.int32, (L, 1, 1), 0) % W
    mask_l = (wco != 0).astype(f32)
    mask_r = (wco != W - 1).astype(f32)
    acc = jnp.zeros((L, bblk, mid), f32)
    for dh in (-1, 0, 1):
        for dw in (-1, 0, 1):
            t = 3 * (dh + 1) + (dw + 1)
            tap = hp[29 + dh * W + dw:29 + dh * W + dw + L]
            if dw == -1:
                tap = tap * mask_l
            elif dw == 1:
                tap = tap * mask_r
            acc = acc + tap * pk_ref[4 + t:5 + t, :mid].reshape(1, 1, mid)
    h2 = acc + pk_ref[1:2, :mid].reshape(1, 1, mid)

    out = lax.dot_general(h2, w3_ref[...], dn, preferred_element_type=f32)
    no = C - half
    out = jnp.maximum(out + pk_ref[3:4, :no].reshape(1, 1, no), 0.0)
    o_ref[...] = jnp.concatenate([o_left, out], axis=2)


def _bn_fold(gamma, beta, mean, var):
    s = gamma * lax.rsqrt(var + _EPS)
    return s, beta - mean * s


def kernel(x, channel_choice, bn1_beta, bn1_gamma, bn1_mean, bn1_var,
           bn2_beta, bn2_gamma, bn2_mean, bn2_var,
           bn3_beta, bn3_gamma, bn3_mean, bn3_var,
           w1, w3, wd):
    B, C, H, W = x.shape
    mid = w1.shape[0]
    outputs = w3.shape[0]
    half = C // 2
    L = H * W

    s1, b1 = _bn_fold(bn1_gamma, bn1_beta, bn1_mean, bn1_var)
    s2, b2 = _bn_fold(bn2_gamma, bn2_beta, bn2_mean, bn2_var)
    s3, b3 = _bn_fold(bn3_gamma, bn3_beta, bn3_mean, bn3_var)

    mask = channel_choice[0, :mid]
    w1_eff = w1 * (mask * s1)[:, None]

    pk = jnp.concatenate([
        jnp.stack([b1, b2, s3, b3]),
        wd * s2[None, :],
        jnp.zeros((3, mid), jnp.float32)], axis=0)
    w3_t = (w3 * s3[:, None]).T

    bblk = 8
    xt = x.transpose(2, 3, 0, 1).reshape(L, B, C)
    kern = functools.partial(_fused_block_kernel, half=half, mid=mid, H=H,
                             W=W, bblk=bblk)
    out = pl.pallas_call(
        kern,
        out_shape=jax.ShapeDtypeStruct((L, B, half + outputs), jnp.float32),
        grid_spec=pltpu.PrefetchScalarGridSpec(
            num_scalar_prefetch=0,
            grid=(B // bblk,),
            in_specs=[
                pl.BlockSpec((L, bblk, C), lambda b: (0, b, 0)),
                pl.BlockSpec((mid, half), lambda b: (0, 0)),
                pl.BlockSpec((16, mid), lambda b: (0, 0)),
                pl.BlockSpec((mid, outputs), lambda b: (0, 0)),
            ],
            out_specs=pl.BlockSpec((L, bblk, half + outputs),
                                   lambda b: (0, b, 0)),
        ),
        compiler_params=pltpu.CompilerParams(
            dimension_semantics=("parallel",),
            vmem_limit_bytes=_VMEM_LIMIT,
        ),
    )(xt, w1_eff, pk, w3_t)
    return out.reshape(H, W, B, half + outputs).transpose(2, 3, 0, 1)


# speedup vs baseline: 1.0803x; 1.0761x over previous
import functools

import jax
import jax.numpy as jnp
from jax import lax
from jax.experimental import pallas as pl
from jax.experimental.pallas import tpu as pltpu

_EPS = 1e-5
_VMEM_LIMIT = 100 * 1024 * 1024


def _fused_block_kernel(x_ref, w1_ref, pk_ref, w3_ref, o_ref, *,
                        half, mid, H, W, bblk):
    L = H * W
    C = x_ref.shape[2]
    f32 = jnp.float32

    row = lax.broadcasted_iota(jnp.int32, (C, half), 0)
    col = lax.broadcasted_iota(jnp.int32, (C, half), 1)
    left = (row == 2 * col).astype(f32)
    w1t = w1_ref[...].T * pk_ref[4:5, :mid]
    right = jnp.stack([jnp.zeros((half, mid), f32), w1t],
                      axis=1).reshape(C, mid)
    big_w = jnp.concatenate([left, right], axis=1)

    dn = (((2,), (0,)), ((), ()))
    y = lax.dot_general(x_ref[...], big_w, dn,
                        preferred_element_type=f32)
    o_left = y[:, :, :half]
    b1 = pk_ref[0:1, :mid].reshape(1, 1, mid)
    h1 = jnp.maximum(y[:, :, half:] + b1, 0.0)

    zp = jnp.zeros((29, bblk, mid), f32)
    hp = jnp.concatenate([zp, h1, zp], axis=0)
    wco = lax.broadcasted_iota(jnp.int32, (L, 1, 1), 0) % W
    mask_l = (wco != 0).astype(f32)
    mask_r = (wco != W - 1).astype(f32)
    acc = jnp.zeros((L, bblk, mid), f32)
    for dh in (-1, 0, 1):
        for dw in (-1, 0, 1):
            t = 3 * (dh + 1) + (dw + 1)
            tap = hp[29 + dh * W + dw:29 + dh * W + dw + L]
            if dw == -1:
                tap = tap * mask_l
            elif dw == 1:
                tap = tap * mask_r
            acc = acc + tap * pk_ref[5 + t:6 + t, :mid].reshape(1, 1, mid)
    h2 = acc + pk_ref[1:2, :mid].reshape(1, 1, mid)

    no = x_ref.shape[2] - half
    w3t = w3_ref[...].T * pk_ref[2:3, :no]
    out = lax.dot_general(h2, w3t, dn, preferred_element_type=f32)
    no = C - half
    out = jnp.maximum(out + pk_ref[3:4, :no].reshape(1, 1, no), 0.0)
    o_ref[...] = jnp.concatenate([o_left, out], axis=2)


def _bn_fold(gamma, beta, mean, var):
    s = gamma * lax.rsqrt(var + _EPS)
    return s, beta - mean * s


def kernel(x, channel_choice, bn1_beta, bn1_gamma, bn1_mean, bn1_var,
           bn2_beta, bn2_gamma, bn2_mean, bn2_var,
           bn3_beta, bn3_gamma, bn3_mean, bn3_var,
           w1, w3, wd):
    B, C, H, W = x.shape
    mid = w1.shape[0]
    outputs = w3.shape[0]
    half = C // 2
    L = H * W

    s1, b1 = _bn_fold(bn1_gamma, bn1_beta, bn1_mean, bn1_var)
    s2, b2 = _bn_fold(bn2_gamma, bn2_beta, bn2_mean, bn2_var)
    s3, b3 = _bn_fold(bn3_gamma, bn3_beta, bn3_mean, bn3_var)

    mask = channel_choice[0, :mid]

    pk = jnp.concatenate([
        jnp.stack([b1, b2, s3, b3, mask * s1]),
        wd * s2[None, :],
        jnp.zeros((2, mid), jnp.float32)], axis=0)

    bblk = 8
    xt = x.transpose(2, 3, 0, 1).reshape(L, B, C)
    kern = functools.partial(_fused_block_kernel, half=half, mid=mid, H=H,
                             W=W, bblk=bblk)
    out = pl.pallas_call(
        kern,
        out_shape=jax.ShapeDtypeStruct((L, B, half + outputs), jnp.float32),
        grid_spec=pltpu.PrefetchScalarGridSpec(
            num_scalar_prefetch=0,
            grid=(B // bblk,),
            in_specs=[
                pl.BlockSpec((L, bblk, C), lambda b: (0, b, 0)),
                pl.BlockSpec((mid, half), lambda b: (0, 0)),
                pl.BlockSpec((16, mid), lambda b: (0, 0)),
                pl.BlockSpec((outputs, mid), lambda b: (0, 0)),
            ],
            out_specs=pl.BlockSpec((L, bblk, half + outputs),
                                   lambda b: (0, b, 0)),
        ),
        compiler_params=pltpu.CompilerParams(
            dimension_semantics=("parallel",),
            vmem_limit_bytes=_VMEM_LIMIT,
        ),
    )(xt, w1, pk, w3)
    return out.reshape(H, W, B, half + outputs).transpose(2, 3, 0, 1)


# speedup vs baseline: 1.2318x; 1.1402x over previous
import functools

import jax
import jax.numpy as jnp
from jax import lax
from jax.experimental import pallas as pl
from jax.experimental.pallas import tpu as pltpu

_EPS = 1e-5
_VMEM_LIMIT = 100 * 1024 * 1024


def _fused_block_kernel(x_ref, cc_ref,
                        b1b_ref, b1g_ref, b1m_ref, b1v_ref,
                        b2b_ref, b2g_ref, b2m_ref, b2v_ref,
                        b3b_ref, b3g_ref, b3m_ref, b3v_ref,
                        w1_ref, w3_ref, wd_ref, o_ref, *,
                        half, mid, H, W, bblk):
    L = H * W
    C = x_ref.shape[2]
    f32 = jnp.float32

    def bn_fold(beta_ref, gamma_ref, mean_ref, var_ref):
        v = var_ref[...] + _EPS
        r = lax.rsqrt(v)
        r = r * (1.5 - 0.5 * v * r * r)
        s = gamma_ref[...] * r
        return s, beta_ref[...] - mean_ref[...] * s

    s1, b1 = bn_fold(b1b_ref, b1g_ref, b1m_ref, b1v_ref)
    s2, b2 = bn_fold(b2b_ref, b2g_ref, b2m_ref, b2v_ref)
    s3, b3 = bn_fold(b3b_ref, b3g_ref, b3m_ref, b3v_ref)

    row = lax.broadcasted_iota(jnp.int32, (C, half), 0)
    col = lax.broadcasted_iota(jnp.int32, (C, half), 1)
    left = (row == 2 * col).astype(f32)
    w1t = w1_ref[...].T * (cc_ref[...][:, :mid] * s1)
    right = jnp.stack([jnp.zeros((half, mid), f32), w1t],
                      axis=1).reshape(C, mid)
    big_w = jnp.concatenate([left, right], axis=1)

    dn = (((2,), (0,)), ((), ()))
    y = lax.dot_general(x_ref[...], big_w, dn,
                        preferred_element_type=f32)
    o_left = y[:, :, :half]
    h1 = jnp.maximum(y[:, :, half:] + b1.reshape(1, 1, mid), 0.0)

    wde = wd_ref[...] * s2
    zp = jnp.zeros((29, bblk, mid), f32)
    hp = jnp.concatenate([zp, h1, zp], axis=0)
    wco = lax.broadcasted_iota(jnp.int32, (L, 1, 1), 0) % W
    mask_l = (wco != 0).astype(f32)
    mask_r = (wco != W - 1).astype(f32)
    acc = jnp.zeros((L, bblk, mid), f32)
    for dh in (-1, 0, 1):
        for dw in (-1, 0, 1):
            t = 3 * (dh + 1) + (dw + 1)
            tap = hp[29 + dh * W + dw:29 + dh * W + dw + L]
            if dw == -1:
                tap = tap * mask_l
            elif dw == 1:
                tap = tap * mask_r
            acc = acc + tap * wde[t:t + 1, :].reshape(1, 1, mid)
    h2 = acc + b2.reshape(1, 1, mid)

    no = C - half
    w3t = w3_ref[...].T * s3[:, :no]
    out = lax.dot_general(h2, w3t, dn, preferred_element_type=f32)
    out = jnp.maximum(out + b3.reshape(1, 1, no), 0.0)
    o_ref[...] = jnp.concatenate([o_left, out], axis=2)


def kernel(x, channel_choice, bn1_beta, bn1_gamma, bn1_mean, bn1_var,
           bn2_beta, bn2_gamma, bn2_mean, bn2_var,
           bn3_beta, bn3_gamma, bn3_mean, bn3_var,
           w1, w3, wd):
    B, C, H, W = x.shape
    mid = w1.shape[0]
    outputs = w3.shape[0]
    half = C // 2
    L = H * W

    bblk = 8
    xt = x.transpose(2, 3, 0, 1).reshape(L, B, C)
    rows = [v.reshape(1, -1) for v in
            (bn1_beta, bn1_gamma, bn1_mean, bn1_var,
             bn2_beta, bn2_gamma, bn2_mean, bn2_var,
             bn3_beta, bn3_gamma, bn3_mean, bn3_var)]
    kern = functools.partial(_fused_block_kernel, half=half, mid=mid, H=H,
                             W=W, bblk=bblk)
    row_spec = [pl.BlockSpec(r.shape, lambda b: (0, 0)) for r in rows]
    out = pl.pallas_call(
        kern,
        out_shape=jax.ShapeDtypeStruct((L, B, half + outputs), jnp.float32),
        grid_spec=pltpu.PrefetchScalarGridSpec(
            num_scalar_prefetch=0,
            grid=(B // bblk,),
            in_specs=[
                pl.BlockSpec((L, bblk, C), lambda b: (0, b, 0)),
                pl.BlockSpec(channel_choice.shape, lambda b: (0, 0)),
            ] + row_spec + [
                pl.BlockSpec((mid, half), lambda b: (0, 0)),
                pl.BlockSpec((outputs, mid), lambda b: (0, 0)),
                pl.BlockSpec((wd.shape[0], mid), lambda b: (0, 0)),
            ],
            out_specs=pl.BlockSpec((L, bblk, half + outputs),
                                   lambda b: (0, b, 0)),
        ),
        compiler_params=pltpu.CompilerParams(
            dimension_semantics=("parallel",),
            vmem_limit_bytes=_VMEM_LIMIT,
        ),
    )(xt, channel_choice, *rows, w1, w3, wd)
    return out.reshape(H, W, B, half + outputs).transpose(2, 3, 0, 1)
